# baseline (speedup 1.0000x reference)
import functools

import jax
import jax.numpy as jnp
from jax import lax
from jax.experimental import pallas as pl
from jax.experimental.pallas import tpu as pltpu
from jax.experimental.pallas import tpu_sc as plsc

N = 10000
E = 320000
NC, NS, NL = 2, 16, 16
NW = NC * NS
EW = E // NW
NBLK = EW // NL
NBLKP = 628
EWP = NBLKP * NL
SUB_ROWS = 632
N_PAD = SUB_ROWS * NS


def _shuffle(v, idxvec):
    dn = lax.GatherDimensionNumbers(
        offset_dims=(), collapsed_slice_dims=(0,), start_index_map=(0,))
    return lax.gather(v, idxvec.reshape(NL, 1), dn, (1,),
                      mode=lax.GatherScatterMode.PROMISE_IN_BOUNDS)


def _bcast_lane(v, r):
    return _shuffle(v, jnp.full((NL,), r, jnp.int32))


@functools.cache
def _make_edge_pass(hp, d, interpret=False):
    dh = d // hp
    mesh = plsc.VectorSubcoreMesh(core_axis_name="c", subcore_axis_name="s",
                                  num_cores=NC, num_subcores=NS)

    @functools.partial(
        pl.kernel,
        out_type=[
            jax.ShapeDtypeStruct((NC, N_PAD, d), jnp.float32),
            jax.ShapeDtypeStruct((NC, N_PAD, 16), jnp.float32),
        ],
        mesh=mesh,
        scratch_types=[
            pltpu.VMEM_SHARED((N_PAD, d), jnp.float32),
            pltpu.VMEM_SHARED((N_PAD, 16), jnp.float32),
            pltpu.VMEM((EWP,), jnp.int32),
            pltpu.VMEM((EWP,), jnp.int32),
            [pltpu.VMEM((NL, d), jnp.float32)] * 2,
            [pltpu.VMEM((NL, d), jnp.float32)] * 2,
            [pltpu.VMEM((NL, 16), jnp.float32)] * 2,
            [pltpu.VMEM((NL, 16), jnp.float32)] * 2,
            [pltpu.VMEM((NL, 16), jnp.float32)] * 2,
            [pltpu.SemaphoreType.DMA] * 2,
            [pltpu.SemaphoreType.DMA] * 2,
        ],
        compiler_params=pltpu.CompilerParams(needs_layout_passes=False,
                                             use_tc_tiling_on_sc=False),
        interpret=interpret,
    )
    def edge_pass(src_h, dst_h, sst_h, ddt_h, v_h, acc_h, den_h,
                  acc_s, den_s, src_v, dst_v,
                  vbuf, tbuf, sbuf, dbuf, denrows, gsem, ssem):
        cid = lax.axis_index("c")
        sid = lax.axis_index("s")
        wid = sid * NC + cid

        zero = jnp.zeros((NL,), jnp.float32)
        for q in range(2):
            for r in range(NL):
                for j in range(d // NL):
                    tbuf[q][r, pl.ds(j * NL, NL)] = zero
                denrows[q][r, :] = zero
        row0 = sid * SUB_ROWS

        def zero_body(k, _):
            pltpu.sync_copy(tbuf[0].at[pl.ds(0, 8)],
                            acc_s.at[pl.ds(row0 + k * 8, 8)])
            pltpu.sync_copy(denrows[0].at[pl.ds(0, 8)],
                            den_s.at[pl.ds(row0 + k * 8, 8)])
            return _
        lax.fori_loop(0, SUB_ROWS // 8, zero_body, None)

        pltpu.sync_copy(src_h.at[pl.ds(wid * EWP, EWP)], src_v)
        pltpu.sync_copy(dst_h.at[pl.ds(wid * EWP, EWP)], dst_v)

        plsc.subcore_barrier()

        lanes = lax.iota(jnp.int32, NL)
        lt_hp = lanes < hp
        rot = hp + (lanes % hp if hp > 1 else lanes * 0)

        def start_gathers(b, q):
            sidx = src_v[pl.ds(b * NL, NL)]
            didx = dst_v[pl.ds(b * NL, NL)]
            pltpu.make_async_copy(v_h.at[sidx], vbuf[q], gsem[q]).start()
            pltpu.make_async_copy(sst_h.at[sidx], sbuf[q], gsem[q]).start()
            pltpu.make_async_copy(ddt_h.at[didx], dbuf[q], gsem[q]).start()

        def wait_gathers(q):
            sidx = src_v[pl.ds(0, NL)]
            pltpu.make_async_copy(v_h.at[sidx], vbuf[q], gsem[q]).wait()
            pltpu.make_async_copy(sst_h.at[sidx], sbuf[q], gsem[q]).wait()
            pltpu.make_async_copy(ddt_h.at[sidx], dbuf[q], gsem[q]).wait()

        def start_scatters(b, q):
            didx = dst_v[pl.ds(b * NL, NL)]
            pltpu.make_async_copy(tbuf[q], acc_s.at[didx],
                                  ssem[q]).start(add=True)
            pltpu.make_async_copy(denrows[q], den_s.at[didx],
                                  ssem[q]).start(add=True)

        def wait_scatters(q):
            didx = dst_v[pl.ds(0, NL)]
            pltpu.make_async_copy(tbuf[q], acc_s.at[didx], ssem[q]).wait()
            pltpu.make_async_copy(denrows[q], den_s.at[didx], ssem[q]).wait()

        def compute(q):
            for r in range(NL):
                srow = sbuf[q][r, :]
                drow = dbuf[q][r, :]
                e = srow + drow
                e = jnp.where(e >= 0, e, 0.2 * e)
                w = jnp.exp(e - _shuffle(drow, rot))
                w = jnp.where(lt_hp, w, 0.0)
                denrows[q][r, :] = w
                for h in range(hp):
                    wb = _bcast_lane(w, h)
                    for j in range(dh // NL):
                        sl = pl.ds(h * dh + j * NL, NL)
                        tbuf[q][r, sl] = wb * vbuf[q][r, sl]

        start_gathers(0, 0)
        start_scatters(0, 0)
        start_scatters(0, 1)

        def pair_body(i, _):
            b0 = i * 2
            start_gathers(b0 + 1, 1)
            wait_gathers(0)
            wait_scatters(0)
            compute(0)
            start_scatters(b0, 0)
            start_gathers(b0 + 2, 0)
            wait_gathers(1)
            wait_scatters(1)
            compute(1)
            start_scatters(b0 + 1, 1)
            return _

        lax.fori_loop(0, 313, pair_body, None)

        wait_gathers(0)
        wait_scatters(0)
        wait_scatters(1)

        plsc.subcore_barrier()
        pltpu.sync_copy(acc_s.at[pl.ds(row0, SUB_ROWS)],
                        acc_h.at[cid, pl.ds(row0, SUB_ROWS)])
        pltpu.sync_copy(den_s.at[pl.ds(row0, SUB_ROWS)],
                        den_h.at[cid, pl.ds(row0, SUB_ROWS)])

    return edge_pass


def _lrelu(x):
    return jnp.where(x >= 0, x, 0.2 * x)


BN = 2000


@functools.cache
def _make_proj_scores(din, dout):
    def body(x_ref, w_ref, asd_ref, h_ref, sc_ref, smax_ref):
        i = pl.program_id(0)
        h = jnp.dot(x_ref[...], w_ref[...], preferred_element_type=jnp.float32)
        h_ref[...] = h
        sc = jnp.dot(h, asd_ref[...], preferred_element_type=jnp.float32)
        sc_ref[...] = sc
        m = jnp.max(sc, axis=0, keepdims=True)

        @pl.when(i == 0)
        def _():
            smax_ref[...] = m

        @pl.when(i > 0)
        def _():
            smax_ref[...] = jnp.maximum(smax_ref[...], m)

    return pl.pallas_call(
        body,
        grid=(N // BN,),
        in_specs=[
            pl.BlockSpec((BN, din), lambda i: (i, 0)),
            pl.BlockSpec((din, dout), lambda i: (0, 0)),
            pl.BlockSpec((dout, 128), lambda i: (0, 0)),
        ],
        out_specs=[
            pl.BlockSpec((BN, dout), lambda i: (i, 0)),
            pl.BlockSpec((BN, 128), lambda i: (i, 0)),
            pl.BlockSpec((1, 128), lambda i: (0, 0)),
        ],
        out_shape=[
            jax.ShapeDtypeStruct((N, dout), jnp.float32),
            jax.ShapeDtypeStruct((N, 128), jnp.float32),
            jax.ShapeDtypeStruct((1, 128), jnp.float32),
        ],
    )


@functools.cache
def _make_tables(H, n_sc):
    def body(*refs):
        sc_refs, smax_refs, tb_ref = refs[:n_sc], refs[n_sc:2 * n_sc], refs[-1]
        sc = sc_refs[0][...]
        smax = smax_refs[0][...]
        for k in range(1, n_sc):
            sc = sc + sc_refs[k][...]
            smax = smax + smax_refs[k][...]
        col = lax.broadcasted_iota(jnp.int32, (BN, 16), 1)
        ss = sc[:, 0:16]
        sd = sc[:, H:H + 16]
        c = _lrelu(smax[:, 0:16] + sd)
        wself = jnp.exp(_lrelu(ss + sd) - c)
        msk = col < H
        z = jnp.zeros((BN, 16), jnp.float32)
        tb_ref[:, 0:16] = jnp.where(msk, ss, z)
        tb_ref[:, 16:32] = jnp.where(msk, sd, z)
        tb_ref[:, 32:48] = jnp.where(msk, c, z)
        tb_ref[:, 48:64] = jnp.where(msk, wself, z)
        tb_ref[:, 64:128] = jnp.zeros((BN, 64), jnp.float32)

    return pl.pallas_call(
        body,
        grid=(N // BN,),
        in_specs=[pl.BlockSpec((BN, 128), lambda i: (i, 0))] * n_sc
        + [pl.BlockSpec((1, 128), lambda i: (0, 0))] * n_sc,
        out_specs=pl.BlockSpec((BN, 128), lambda i: (i, 0)),
        out_shape=jax.ShapeDtypeStruct((N, 128), jnp.float32),
    )


@functools.cache
def _make_combine(d, hp, act, ho=0):
    dh = d // hp

    def body(a0_ref, a1_ref, d0_ref, d1_ref, ws_ref, v_ref, b_ref, o_ref):
        num = a0_ref[...] + a1_ref[...]
        outs = []
        for h in range(hp):
            w = ws_ref[:, ho + h:ho + h + 1]
            dent = (d0_ref[:, h:h + 1] + d1_ref[:, h:h + 1] + w + 1e-16)
            sl = slice(h * dh, (h + 1) * dh)
            outs.append((num[:, sl] + w * v_ref[:, sl]) / dent)
        agg = jnp.concatenate(outs, axis=1) if hp > 1 else outs[0]
        agg = agg + b_ref[...]
        if act:
            agg = jnp.where(agg > 0, agg, jnp.exp(jnp.minimum(agg, 0.0)) - 1.0)
        o_ref[...] = agg

    return pl.pallas_call(
        body,
        grid=(N // BN,),
        in_specs=[
            pl.BlockSpec((BN, d), lambda i: (i, 0)),
            pl.BlockSpec((BN, d), lambda i: (i, 0)),
            pl.BlockSpec((BN, 16), lambda i: (i, 0)),
            pl.BlockSpec((BN, 16), lambda i: (i, 0)),
            pl.BlockSpec((BN, 16), lambda i: (i, 0)),
            pl.BlockSpec((BN, d), lambda i: (i, 0)),
            pl.BlockSpec((1, d), lambda i: (0, 0)),
        ],
        out_specs=pl.BlockSpec((BN, d), lambda i: (i, 0)),
        out_shape=jax.ShapeDtypeStruct((N, d), jnp.float32),
    )


def _pack_tables(tb, hp, ho):
    sst = jnp.zeros((N_PAD, 16), jnp.float32)
    sst = sst.at[:N, 0:hp].set(tb[:, ho:ho + hp])
    ddt = jnp.zeros((N_PAD, 16), jnp.float32)
    ddt = ddt.at[:N, 0:hp].set(tb[:, 16 + ho:16 + ho + hp])
    ddt = ddt.at[:N, hp:2 * hp].set(tb[:, 32 + ho:32 + ho + hp])
    return sst, ddt


def _place_cols(vecs, nrows):
    m = jnp.zeros((nrows, 128), jnp.float32)
    for vec, r0, col in vecs:
        m = m.at[r0:r0 + vec.shape[0], col].set(vec)
    return m


def kernel(x, edge_index, W1, as1, ad1, b1, W2, as2, ad2, b2,
           Wmu, asmu, admu, bmu, Wls, asls, adls, bls):
    pad_s = jnp.zeros((NW, EWP - EW), jnp.int32)
    pad_d = jnp.full((NW, EWP - EW), N, jnp.int32)
    src = jnp.concatenate([edge_index[0].reshape(NW, EW), pad_s],
                          axis=1).reshape(-1)
    dst = jnp.concatenate([edge_index[1].reshape(NW, EW), pad_d],
                          axis=1).reshape(-1)

    asd1 = _place_cols([(as1[h], 64 * h, h) for h in range(8)]
                       + [(ad1[h], 64 * h, 8 + h) for h in range(8)], 512)
    h1, sc1, smax1 = _make_proj_scores(128, 512)(x, W1, asd1)
    tb1 = _make_tables(8, 1)(sc1, smax1)
    outs = []
    for p in range(4):
        vp = h1[:, 128 * p:128 * (p + 1)]
        sst, ddt = _pack_tables(tb1, 2, 2 * p)
        acc, den = _make_edge_pass(2, 128)(src, dst, sst, ddt, vp)
        outs.append(_make_combine(128, 2, True, 2 * p)(
            acc[0, :N], acc[1, :N], den[0, :N], den[1, :N],
            tb1[:, 48:64], vp, b1[128 * p:128 * (p + 1)].reshape(1, 128)))
    h2in = jnp.concatenate(outs, axis=1)

    w2p = jnp.zeros((512, 128), jnp.float32).at[:, 0:64].set(W2)
    asd2 = _place_cols([(as2[0], 0, 0), (ad2[0], 0, 1)], 128)
    h2p, sc2, smax2 = _make_proj_scores(512, 128)(h2in, w2p, asd2)
    v2 = h2p[:, 0:64]
    sst2, ddt2 = _pack_tables(tb2 := _make_tables(1, 1)(sc2, smax2), 1, 0)
    acc2, den2 = _make_edge_pass(1, 64)(src, dst, sst2, ddt2, v2)
    h3 = _make_combine(64, 1, True)(
        acc2[0, :N], acc2[1, :N], den2[0, :N], den2[1, :N],
        tb2[:, 48:64], v2, b2.reshape(1, 64))

    wmup = jnp.zeros((64, 128), jnp.float32).at[:, 0:32].set(Wmu)
    wlsp = jnp.zeros((64, 128), jnp.float32).at[:, 32:64].set(Wls)
    asdmu = _place_cols([(asmu[0], 0, 0), (admu[0], 0, 2)], 128)
    asdls = _place_cols([(asls[0], 32, 1), (adls[0], 32, 3)], 128)
    hmu, scmu, smaxmu = _make_proj_scores(64, 128)(h3, wmup, asdmu)
    hls, scls, smaxls = _make_proj_scores(64, 128)(h3, wlsp, asdls)
    vml = jnp.concatenate([hmu[:, 0:32], hls[:, 32:64]], axis=1)
    tbml = _make_tables(2, 2)(scmu, scls, smaxmu, smaxls)
    sstml, ddtml = _pack_tables(tbml, 2, 0)
    accml, denml = _make_edge_pass(2, 64)(src, dst, sstml, ddtml, vml)
    bml = jnp.concatenate([bmu, bls]).reshape(1, 64)
    outml = _make_combine(64, 2, False)(
        accml[0, :N], accml[1, :N], denml[0, :N], denml[1, :N],
        tbml[:, 48:64], vml, bml)
    return (outml[:, :32], outml[:, 32:])


# speedup vs baseline: 1.0709x; 1.0709x over previous
import functools

import jax
import jax.numpy as jnp
from jax import lax
from jax.experimental import pallas as pl
from jax.experimental.pallas import tpu as pltpu
from jax.experimental.pallas import tpu_sc as plsc

N = 10000
E = 320000
NC, NS, NL = 2, 16, 16
NW = NC * NS
EW = E // NW
NBLK = EW // NL
NBLKP = 632
EWP = NBLKP * NL
SUB_ROWS = 632
N_PAD = SUB_ROWS * NS


def _shuffle(v, idxvec):
    dn = lax.GatherDimensionNumbers(
        offset_dims=(), collapsed_slice_dims=(0,), start_index_map=(0,))
    return lax.gather(v, idxvec.reshape(NL, 1), dn, (1,),
                      mode=lax.GatherScatterMode.PROMISE_IN_BOUNDS)


def _bcast_lane(v, r):
    return _shuffle(v, jnp.full((NL,), r, jnp.int32))


@functools.cache
def _make_edge_pass(hp, d, interpret=False):
    dh = d // hp
    mesh = plsc.VectorSubcoreMesh(core_axis_name="c", subcore_axis_name="s",
                                  num_cores=NC, num_subcores=NS)

    @functools.partial(
        pl.kernel,
        out_type=[
            jax.ShapeDtypeStruct((NC, N_PAD, d), jnp.float32),
            jax.ShapeDtypeStruct((NC, N_PAD, 16), jnp.float32),
        ],
        mesh=mesh,
        scratch_types=[
            pltpu.VMEM_SHARED((N_PAD, d), jnp.float32),
            pltpu.VMEM_SHARED((N_PAD, 16), jnp.float32),
            pltpu.VMEM((EWP,), jnp.int32),
            pltpu.VMEM((EWP,), jnp.int32),
            [pltpu.VMEM((NL, d), jnp.float32)] * 3,
            [pltpu.VMEM((NL, d), jnp.float32)] * 3,
            [pltpu.VMEM((NL, 16), jnp.float32)] * 3,
            [pltpu.VMEM((NL, 16), jnp.float32)] * 3,
            [pltpu.VMEM((NL, 16), jnp.float32)] * 3,
            [pltpu.SemaphoreType.DMA] * 3,
            [pltpu.SemaphoreType.DMA] * 3,
        ],
        compiler_params=pltpu.CompilerParams(needs_layout_passes=False,
                                             use_tc_tiling_on_sc=False),
        interpret=interpret,
    )
    def edge_pass(src_h, dst_h, sst_h, ddt_h, v_h, acc_h, den_h,
                  acc_s, den_s, src_v, dst_v,
                  vbuf, tbuf, sbuf, dbuf, denrows, gsem, ssem):
        cid = lax.axis_index("c")
        sid = lax.axis_index("s")
        wid = sid * NC + cid

        zero = jnp.zeros((NL,), jnp.float32)
        for q in range(3):
            for r in range(NL):
                for j in range(d // NL):
                    tbuf[q][r, pl.ds(j * NL, NL)] = zero
                denrows[q][r, :] = zero
        row0 = sid * SUB_ROWS

        def zero_body(k, _):
            pltpu.sync_copy(tbuf[0].at[pl.ds(0, 8)],
                            acc_s.at[pl.ds(row0 + k * 8, 8)])
            pltpu.sync_copy(denrows[0].at[pl.ds(0, 8)],
                            den_s.at[pl.ds(row0 + k * 8, 8)])
            return _
        lax.fori_loop(0, SUB_ROWS // 8, zero_body, None)

        pltpu.sync_copy(src_h.at[pl.ds(wid * EWP, EWP)], src_v)
        pltpu.sync_copy(dst_h.at[pl.ds(wid * EWP, EWP)], dst_v)

        plsc.subcore_barrier()

        lanes = lax.iota(jnp.int32, NL)
        lt_hp = lanes < hp
        rot = hp + (lanes % hp if hp > 1 else lanes * 0)

        def start_gathers(b, q):
            sidx = src_v[pl.ds(b * NL, NL)]
            didx = dst_v[pl.ds(b * NL, NL)]
            pltpu.make_async_copy(v_h.at[sidx], vbuf[q], gsem[q]).start()
            pltpu.make_async_copy(sst_h.at[sidx], sbuf[q], gsem[q]).start()
            pltpu.make_async_copy(ddt_h.at[didx], dbuf[q], gsem[q]).start()

        def wait_gathers(q):
            sidx = src_v[pl.ds(0, NL)]
            pltpu.make_async_copy(v_h.at[sidx], vbuf[q], gsem[q]).wait()
            pltpu.make_async_copy(sst_h.at[sidx], sbuf[q], gsem[q]).wait()
            pltpu.make_async_copy(ddt_h.at[sidx], dbuf[q], gsem[q]).wait()

        def start_scatters(b, q):
            didx = dst_v[pl.ds(b * NL, NL)]
            pltpu.make_async_copy(tbuf[q], acc_s.at[didx],
                                  ssem[q]).start(add=True)
            pltpu.make_async_copy(denrows[q], den_s.at[didx],
                                  ssem[q]).start(add=True)

        def wait_scatters(q):
            didx = dst_v[pl.ds(0, NL)]
            pltpu.make_async_copy(tbuf[q], acc_s.at[didx], ssem[q]).wait()
            pltpu.make_async_copy(denrows[q], den_s.at[didx], ssem[q]).wait()

        def compute(q):
            for r in range(NL):
                srow = sbuf[q][r, :]
                drow = dbuf[q][r, :]
                e = srow + drow
                e = jnp.where(e >= 0, e, 0.2 * e)
                w = jnp.exp(e - _shuffle(drow, rot))
                w = jnp.where(lt_hp, w, 0.0)
                denrows[q][r, :] = w
                for h in range(hp):
                    wb = _bcast_lane(w, h)
                    for j in range(dh // NL):
                        sl = pl.ds(h * dh + j * NL, NL)
                        tbuf[q][r, sl] = wb * vbuf[q][r, sl]

        for q in range(3):
            start_gathers(q, q)
            start_scatters(0, q)

        def tri_body(i, _):
            b0 = i * 3
            for q in range(3):
                wait_gathers(q)
                wait_scatters(q)
                compute(q)
                start_scatters(b0 + q, q)
                start_gathers(b0 + q + 3, q)
            return _

        lax.fori_loop(0, 209, tri_body, None)

        for q in range(3):
            wait_gathers(q)
            wait_scatters(q)

        plsc.subcore_barrier()
        pltpu.sync_copy(acc_s.at[pl.ds(row0, SUB_ROWS)],
                        acc_h.at[cid, pl.ds(row0, SUB_ROWS)])
        pltpu.sync_copy(den_s.at[pl.ds(row0, SUB_ROWS)],
                        den_h.at[cid, pl.ds(row0, SUB_ROWS)])

    return edge_pass


def _lrelu(x):
    return jnp.where(x >= 0, x, 0.2 * x)


BN = 2000


@functools.cache
def _make_proj_scores(din, dout):
    def body(x_ref, w_ref, asd_ref, h_ref, sc_ref, smax_ref):
        i = pl.program_id(0)
        h = jnp.dot(x_ref[...], w_ref[...], preferred_element_type=jnp.float32)
        h_ref[...] = h
        sc = jnp.dot(h, asd_ref[...], preferred_element_type=jnp.float32)
        sc_ref[...] = sc
        m = jnp.max(sc, axis=0, keepdims=True)

        @pl.when(i == 0)
        def _():
            smax_ref[...] = m

        @pl.when(i > 0)
        def _():
            smax_ref[...] = jnp.maximum(smax_ref[...], m)

    return pl.pallas_call(
        body,
        grid=(N // BN,),
        in_specs=[
            pl.BlockSpec((BN, din), lambda i: (i, 0)),
            pl.BlockSpec((din, dout), lambda i: (0, 0)),
            pl.BlockSpec((dout, 128), lambda i: (0, 0)),
        ],
        out_specs=[
            pl.BlockSpec((BN, dout), lambda i: (i, 0)),
            pl.BlockSpec((BN, 128), lambda i: (i, 0)),
            pl.BlockSpec((1, 128), lambda i: (0, 0)),
        ],
        out_shape=[
            jax.ShapeDtypeStruct((N, dout), jnp.float32),
            jax.ShapeDtypeStruct((N, 128), jnp.float32),
            jax.ShapeDtypeStruct((1, 128), jnp.float32),
        ],
    )


@functools.cache
def _make_tables(H, n_sc):
    def body(*refs):
        sc_refs, smax_refs, tb_ref = refs[:n_sc], refs[n_sc:2 * n_sc], refs[-1]
        sc = sc_refs[0][...]
        smax = smax_refs[0][...]
        for k in range(1, n_sc):
            sc = sc + sc_refs[k][...]
            smax = smax + smax_refs[k][...]
        col = lax.broadcasted_iota(jnp.int32, (BN, 16), 1)
        ss = sc[:, 0:16]
        sd = sc[:, H:H + 16]
        c = _lrelu(smax[:, 0:16] + sd)
        wself = jnp.exp(_lrelu(ss + sd) - c)
        msk = col < H
        z = jnp.zeros((BN, 16), jnp.float32)
        tb_ref[:, 0:16] = jnp.where(msk, ss, z)
        tb_ref[:, 16:32] = jnp.where(msk, sd, z)
        tb_ref[:, 32:48] = jnp.where(msk, c, z)
        tb_ref[:, 48:64] = jnp.where(msk, wself, z)
        tb_ref[:, 64:128] = jnp.zeros((BN, 64), jnp.float32)

    return pl.pallas_call(
        body,
        grid=(N // BN,),
        in_specs=[pl.BlockSpec((BN, 128), lambda i: (i, 0))] * n_sc
        + [pl.BlockSpec((1, 128), lambda i: (0, 0))] * n_sc,
        out_specs=pl.BlockSpec((BN, 128), lambda i: (i, 0)),
        out_shape=jax.ShapeDtypeStruct((N, 128), jnp.float32),
    )


@functools.cache
def _make_combine(d, hp, act, ho=0):
    dh = d // hp

    def body(a0_ref, a1_ref, d0_ref, d1_ref, ws_ref, v_ref, b_ref, o_ref):
        num = a0_ref[...] + a1_ref[...]
        outs = []
        for h in range(hp):
            w = ws_ref[:, ho + h:ho + h + 1]
            dent = (d0_ref[:, h:h + 1] + d1_ref[:, h:h + 1] + w + 1e-16)
            sl = slice(h * dh, (h + 1) * dh)
            outs.append((num[:, sl] + w * v_ref[:, sl]) / dent)
        agg = jnp.concatenate(outs, axis=1) if hp > 1 else outs[0]
        agg = agg + b_ref[...]
        if act:
            agg = jnp.where(agg > 0, agg, jnp.exp(jnp.minimum(agg, 0.0)) - 1.0)
        o_ref[...] = agg

    return pl.pallas_call(
        body,
        grid=(N // BN,),
        in_specs=[
            pl.BlockSpec((BN, d), lambda i: (i, 0)),
            pl.BlockSpec((BN, d), lambda i: (i, 0)),
            pl.BlockSpec((BN, 16), lambda i: (i, 0)),
            pl.BlockSpec((BN, 16), lambda i: (i, 0)),
            pl.BlockSpec((BN, 16), lambda i: (i, 0)),
            pl.BlockSpec((BN, d), lambda i: (i, 0)),
            pl.BlockSpec((1, d), lambda i: (0, 0)),
        ],
        out_specs=pl.BlockSpec((BN, d), lambda i: (i, 0)),
        out_shape=jax.ShapeDtypeStruct((N, d), jnp.float32),
    )


def _pack_tables(tb, hp, ho):
    sst = jnp.zeros((N_PAD, 16), jnp.float32)
    sst = sst.at[:N, 0:hp].set(tb[:, ho:ho + hp])
    ddt = jnp.zeros((N_PAD, 16), jnp.float32)
    ddt = ddt.at[:N, 0:hp].set(tb[:, 16 + ho:16 + ho + hp])
    ddt = ddt.at[:N, hp:2 * hp].set(tb[:, 32 + ho:32 + ho + hp])
    return sst, ddt


def _place_cols(vecs, nrows):
    m = jnp.zeros((nrows, 128), jnp.float32)
    for vec, r0, col in vecs:
        m = m.at[r0:r0 + vec.shape[0], col].set(vec)
    return m


def kernel(x, edge_index, W1, as1, ad1, b1, W2, as2, ad2, b2,
           Wmu, asmu, admu, bmu, Wls, asls, adls, bls):
    pad_s = jnp.zeros((NW, EWP - EW), jnp.int32)
    pad_d = jnp.full((NW, EWP - EW), N, jnp.int32)
    src = jnp.concatenate([edge_index[0].reshape(NW, EW), pad_s],
                          axis=1).reshape(-1)
    dst = jnp.concatenate([edge_index[1].reshape(NW, EW), pad_d],
                          axis=1).reshape(-1)

    asd1 = _place_cols([(as1[h], 64 * h, h) for h in range(8)]
                       + [(ad1[h], 64 * h, 8 + h) for h in range(8)], 512)
    h1, sc1, smax1 = _make_proj_scores(128, 512)(x, W1, asd1)
    tb1 = _make_tables(8, 1)(sc1, smax1)
    outs = []
    for p in range(4):
        vp = h1[:, 128 * p:128 * (p + 1)]
        sst, ddt = _pack_tables(tb1, 2, 2 * p)
        acc, den = _make_edge_pass(2, 128)(src, dst, sst, ddt, vp)
        outs.append(_make_combine(128, 2, True, 2 * p)(
            acc[0, :N], acc[1, :N], den[0, :N], den[1, :N],
            tb1[:, 48:64], vp, b1[128 * p:128 * (p + 1)].reshape(1, 128)))
    h2in = jnp.concatenate(outs, axis=1)

    w2p = jnp.zeros((512, 128), jnp.float32).at[:, 0:64].set(W2)
    asd2 = _place_cols([(as2[0], 0, 0), (ad2[0], 0, 1)], 128)
    h2p, sc2, smax2 = _make_proj_scores(512, 128)(h2in, w2p, asd2)
    v2 = h2p[:, 0:64]
    sst2, ddt2 = _pack_tables(tb2 := _make_tables(1, 1)(sc2, smax2), 1, 0)
    acc2, den2 = _make_edge_pass(1, 64)(src, dst, sst2, ddt2, v2)
    h3 = _make_combine(64, 1, True)(
        acc2[0, :N], acc2[1, :N], den2[0, :N], den2[1, :N],
        tb2[:, 48:64], v2, b2.reshape(1, 64))

    wmup = jnp.zeros((64, 128), jnp.float32).at[:, 0:32].set(Wmu)
    wlsp = jnp.zeros((64, 128), jnp.float32).at[:, 32:64].set(Wls)
    asdmu = _place_cols([(asmu[0], 0, 0), (admu[0], 0, 2)], 128)
    asdls = _place_cols([(asls[0], 32, 1), (adls[0], 32, 3)], 128)
    hmu, scmu, smaxmu = _make_proj_scores(64, 128)(h3, wmup, asdmu)
    hls, scls, smaxls = _make_proj_scores(64, 128)(h3, wlsp, asdls)
    vml = jnp.concatenate([hmu[:, 0:32], hls[:, 32:64]], axis=1)
    tbml = _make_tables(2, 2)(scmu, scls, smaxmu, smaxls)
    sstml, ddtml = _pack_tables(tbml, 2, 0)
    accml, denml = _make_edge_pass(2, 64)(src, dst, sstml, ddtml, vml)
    bml = jnp.concatenate([bmu, bls]).reshape(1, 64)
    outml = _make_combine(64, 2, False)(
        accml[0, :N], accml[1, :N], denml[0, :N], denml[1, :N],
        tbml[:, 48:64], vml, bml)
    return (outml[:, :32], outml[:, 32:])


# speedup vs baseline: 1.1134x; 1.0397x over previous
import functools

import jax
import jax.numpy as jnp
from jax import lax
from jax.experimental import pallas as pl
from jax.experimental.pallas import tpu as pltpu
from jax.experimental.pallas import tpu_sc as plsc

N = 10000
E = 320000
NC, NS, NL = 2, 16, 16
NW = NC * NS
EW = E // NW
NBLK = EW // NL
NBLKP = 632
EWP = NBLKP * NL
SUB_ROWS = 632
N_PAD = SUB_ROWS * NS


def _shuffle(v, idxvec):
    dn = lax.GatherDimensionNumbers(
        offset_dims=(), collapsed_slice_dims=(0,), start_index_map=(0,))
    return lax.gather(v, idxvec.reshape(NL, 1), dn, (1,),
                      mode=lax.GatherScatterMode.PROMISE_IN_BOUNDS)


def _bcast_lane(v, r):
    return _shuffle(v, jnp.full((NL,), r, jnp.int32))


@functools.cache
def _make_edge_pass(hp, d, interpret=False):
    dh = d // hp
    dp = d + 16
    mesh = plsc.VectorSubcoreMesh(core_axis_name="c", subcore_axis_name="s",
                                  num_cores=NC, num_subcores=NS)

    @functools.partial(
        pl.kernel,
        out_type=jax.ShapeDtypeStruct((NC, N_PAD, dp), jnp.float32),
        mesh=mesh,
        scratch_types=[
            pltpu.VMEM_SHARED((N_PAD, dp), jnp.float32),
            pltpu.VMEM((EWP,), jnp.int32),
            pltpu.VMEM((EWP,), jnp.int32),
            [pltpu.VMEM((NL, dp), jnp.float32)] * 3,
            [pltpu.VMEM((NL, dp), jnp.float32)] * 3,
            [pltpu.VMEM((NL, 16), jnp.float32)] * 3,
            [pltpu.SemaphoreType.DMA] * 3,
            [pltpu.SemaphoreType.DMA] * 3,
        ],
        compiler_params=pltpu.CompilerParams(needs_layout_passes=False,
                                             use_tc_tiling_on_sc=False),
        interpret=interpret,
    )
    def edge_pass(src_h, dst_h, vs_h, ddt_h, acc_h,
                  acc_s, src_v, dst_v, vbuf, tbuf, dbuf, gsem, ssem):
        cid = lax.axis_index("c")
        sid = lax.axis_index("s")
        wid = sid * NC + cid

        zero = jnp.zeros((NL,), jnp.float32)
        for q in range(3):
            for r in range(NL):
                for j in range(dp // NL):
                    tbuf[q][r, pl.ds(j * NL, NL)] = zero
        row0 = sid * SUB_ROWS

        def zero_body(k, _):
            pltpu.sync_copy(tbuf[0].at[pl.ds(0, 8)],
                            acc_s.at[pl.ds(row0 + k * 8, 8)])
            return _
        lax.fori_loop(0, SUB_ROWS // 8, zero_body, None)

        pltpu.sync_copy(src_h.at[pl.ds(wid * EWP, EWP)], src_v)
        pltpu.sync_copy(dst_h.at[pl.ds(wid * EWP, EWP)], dst_v)

        plsc.subcore_barrier()

        lanes = lax.iota(jnp.int32, NL)
        lt_hp = lanes < hp
        rot = hp + (lanes % hp if hp > 1 else lanes * 0)

        def start_gathers(b, q):
            sidx = src_v[pl.ds(b * NL, NL)]
            didx = dst_v[pl.ds(b * NL, NL)]
            pltpu.make_async_copy(vs_h.at[sidx], vbuf[q], gsem[q]).start()
            pltpu.make_async_copy(ddt_h.at[didx], dbuf[q], gsem[q]).start()

        def wait_gathers(q):
            sidx = src_v[pl.ds(0, NL)]
            pltpu.make_async_copy(vs_h.at[sidx], vbuf[q], gsem[q]).wait()
            pltpu.make_async_copy(ddt_h.at[sidx], dbuf[q], gsem[q]).wait()

        def start_scatters(b, q):
            didx = dst_v[pl.ds(b * NL, NL)]
            pltpu.make_async_copy(tbuf[q], acc_s.at[didx],
                                  ssem[q]).start(add=True)

        def wait_scatters(q):
            didx = dst_v[pl.ds(0, NL)]
            pltpu.make_async_copy(tbuf[q], acc_s.at[didx], ssem[q]).wait()

        def compute(q):
            for r in range(NL):
                srow = vbuf[q][r, pl.ds(d, NL)]
                drow = dbuf[q][r, :]
                e = srow + drow
                e = jnp.where(e >= 0, e, 0.2 * e)
                w = jnp.exp(e - _shuffle(drow, rot))
                w = jnp.where(lt_hp, w, 0.0)
                tbuf[q][r, pl.ds(d, NL)] = w
                for h in range(hp):
                    wb = _bcast_lane(w, h)
                    for j in range(dh // NL):
                        sl = pl.ds(h * dh + j * NL, NL)
                        tbuf[q][r, sl] = wb * vbuf[q][r, sl]

        for q in range(3):
            start_gathers(q, q)
            start_scatters(0, q)

        def tri_body(i, _):
            b0 = i * 3
            for q in range(3):
                wait_gathers(q)
                wait_scatters(q)
                compute(q)
                start_scatters(b0 + q, q)
                start_gathers(b0 + q + 3, q)
            return _

        lax.fori_loop(0, 209, tri_body, None)

        for q in range(3):
            wait_gathers(q)
            wait_scatters(q)

        plsc.subcore_barrier()
        pltpu.sync_copy(acc_s.at[pl.ds(row0, SUB_ROWS)],
                        acc_h.at[cid, pl.ds(row0, SUB_ROWS)])

    return edge_pass


def _lrelu(x):
    return jnp.where(x >= 0, x, 0.2 * x)


BN = 2000


@functools.cache
def _make_proj_scores(din, dout, npass, dch, hp):
    dp = dch + 16

    def body(x_ref, w_ref, asd_ref, *orefs):
        vs_refs, sc_ref, smax_ref = orefs[:npass], orefs[npass], orefs[-1]
        i = pl.program_id(0)
        h = jnp.dot(x_ref[...], w_ref[...], preferred_element_type=jnp.float32)
        sc = jnp.dot(h, asd_ref[...], preferred_element_type=jnp.float32)
        sc_ref[...] = sc
        col = lax.broadcasted_iota(jnp.int32, (BN, 16), 1)
        z = jnp.zeros((BN, 16), jnp.float32)
        for p in range(npass):
            vs_refs[p][:, 0:dch] = h[:, p * dch:(p + 1) * dch]
            ss = sc[:, p * hp:p * hp + 16]
            vs_refs[p][:, dch:dp] = jnp.where(col < hp, ss, z)
        m = jnp.max(sc, axis=0, keepdims=True)

        @pl.when(i == 0)
        def _():
            smax_ref[...] = m

        @pl.when(i > 0)
        def _():
            smax_ref[...] = jnp.maximum(smax_ref[...], m)

    return pl.pallas_call(
        body,
        grid=(N // BN,),
        in_specs=[
            pl.BlockSpec((BN, din), lambda i: (i, 0)),
            pl.BlockSpec((din, dout), lambda i: (0, 0)),
            pl.BlockSpec((dout, 128), lambda i: (0, 0)),
        ],
        out_specs=[pl.BlockSpec((BN, dp), lambda i: (i, 0))] * npass
        + [
            pl.BlockSpec((BN, 128), lambda i: (i, 0)),
            pl.BlockSpec((1, 128), lambda i: (0, 0)),
        ],
        out_shape=[jax.ShapeDtypeStruct((N, dp), jnp.float32)] * npass
        + [
            jax.ShapeDtypeStruct((N, 128), jnp.float32),
            jax.ShapeDtypeStruct((1, 128), jnp.float32),
        ],
    )


@functools.cache
def _make_tables(H, n_sc):
    def body(*refs):
        sc_refs, smax_refs, tb_ref = refs[:n_sc], refs[n_sc:2 * n_sc], refs[-1]
        sc = sc_refs[0][...]
        smax = smax_refs[0][...]
        for k in range(1, n_sc):
            sc = sc + sc_refs[k][...]
            smax = smax + smax_refs[k][...]
        col = lax.broadcasted_iota(jnp.int32, (BN, 16), 1)
        ss = sc[:, 0:16]
        sd = sc[:, H:H + 16]
        c = _lrelu(smax[:, 0:16] + sd)
        wself = jnp.exp(_lrelu(ss + sd) - c)
        msk = col < H
        z = jnp.zeros((BN, 16), jnp.float32)
        tb_ref[:, 0:16] = jnp.where(msk, ss, z)
        tb_ref[:, 16:32] = jnp.where(msk, sd, z)
        tb_ref[:, 32:48] = jnp.where(msk, c, z)
        tb_ref[:, 48:64] = jnp.where(msk, wself, z)
        tb_ref[:, 64:128] = jnp.zeros((BN, 64), jnp.float32)

    return pl.pallas_call(
        body,
        grid=(N // BN,),
        in_specs=[pl.BlockSpec((BN, 128), lambda i: (i, 0))] * n_sc
        + [pl.BlockSpec((1, 128), lambda i: (0, 0))] * n_sc,
        out_specs=pl.BlockSpec((BN, 128), lambda i: (i, 0)),
        out_shape=jax.ShapeDtypeStruct((N, 128), jnp.float32),
    )


@functools.cache
def _make_combine(d, hp, act, ho=0):
    dh = d // hp
    dp = d + 16

    def body(a0_ref, a1_ref, ws_ref, vs_ref, b_ref, o_ref):
        a0 = a0_ref[0]
        a1 = a1_ref[0]
        outs = []
        for h in range(hp):
            w = ws_ref[:, ho + h:ho + h + 1]
            dent = (a0[:, d + h:d + h + 1] + a1[:, d + h:d + h + 1]
                    + w + 1e-16)
            sl = slice(h * dh, (h + 1) * dh)
            outs.append((a0[:, sl] + a1[:, sl] + w * vs_ref[:, sl]) / dent)
        agg = jnp.concatenate(outs, axis=1) if hp > 1 else outs[0]
        agg = agg + b_ref[...]
        if act:
            agg = jnp.where(agg > 0, agg, jnp.exp(jnp.minimum(agg, 0.0)) - 1.0)
        o_ref[...] = agg

    return pl.pallas_call(
        body,
        grid=(N // BN,),
        in_specs=[
            pl.BlockSpec((1, BN, dp), lambda i: (0, i, 0)),
            pl.BlockSpec((1, BN, dp), lambda i: (1, i, 0)),
            pl.BlockSpec((BN, 16), lambda i: (i, 0)),
            pl.BlockSpec((BN, dp), lambda i: (i, 0)),
            pl.BlockSpec((1, d), lambda i: (0, 0)),
        ],
        out_specs=pl.BlockSpec((BN, d), lambda i: (i, 0)),
        out_shape=jax.ShapeDtypeStruct((N, d), jnp.float32),
    )


def _pack_ddt(tb, hp, ho):
    ddt = jnp.zeros((N_PAD, 16), jnp.float32)
    ddt = ddt.at[:N, 0:hp].set(tb[:, 16 + ho:16 + ho + hp])
    ddt = ddt.at[:N, hp:2 * hp].set(tb[:, 32 + ho:32 + ho + hp])
    return ddt


def _place_cols(vecs, nrows):
    m = jnp.zeros((nrows, 128), jnp.float32)
    for vec, r0, col in vecs:
        m = m.at[r0:r0 + vec.shape[0], col].set(vec)
    return m


def kernel(x, edge_index, W1, as1, ad1, b1, W2, as2, ad2, b2,
           Wmu, asmu, admu, bmu, Wls, asls, adls, bls):
    pad_s = jnp.zeros((NW, EWP - EW), jnp.int32)
    pad_d = jnp.full((NW, EWP - EW), N, jnp.int32)
    src = jnp.concatenate([edge_index[0].reshape(NW, EW), pad_s],
                          axis=1).reshape(-1)
    dst = jnp.concatenate([edge_index[1].reshape(NW, EW), pad_d],
                          axis=1).reshape(-1)

    asd1 = _place_cols([(as1[h], 64 * h, h) for h in range(8)]
                       + [(ad1[h], 64 * h, 8 + h) for h in range(8)], 512)
    *vs1, sc1, smax1 = _make_proj_scores(128, 512, 4, 128, 2)(x, W1, asd1)
    tb1 = _make_tables(8, 1)(sc1, smax1)
    outs = []
    for p in range(4):
        ddt = _pack_ddt(tb1, 2, 2 * p)
        acc = _make_edge_pass(2, 128)(src, dst, vs1[p], ddt)
        outs.append(_make_combine(128, 2, True, 2 * p)(
            acc, acc, tb1[:, 48:64], vs1[p],
            b1[128 * p:128 * (p + 1)].reshape(1, 128)))
    h2in = jnp.concatenate(outs, axis=1)

    w2p = jnp.zeros((512, 128), jnp.float32).at[:, 0:64].set(W2)
    asd2 = _place_cols([(as2[0], 0, 0), (ad2[0], 0, 1)], 128)
    vs2, sc2, smax2 = _make_proj_scores(512, 128, 1, 64, 1)(h2in, w2p, asd2)
    tb2 = _make_tables(1, 1)(sc2, smax2)
    acc2 = _make_edge_pass(1, 64)(src, dst, vs2, _pack_ddt(tb2, 1, 0))
    h3 = _make_combine(64, 1, True)(
        acc2, acc2, tb2[:, 48:64], vs2, b2.reshape(1, 64))

    wml = (jnp.zeros((64, 128), jnp.float32)
           .at[:, 0:32].set(Wmu).at[:, 32:64].set(Wls))
    asdml = _place_cols([(asmu[0], 0, 0), (asls[0], 32, 1),
                         (admu[0], 0, 2), (adls[0], 32, 3)], 128)
    vsml, scml, smaxml = _make_proj_scores(64, 128, 1, 64, 2)(h3, wml, asdml)
    tbml = _make_tables(2, 1)(scml, smaxml)
    accml = _make_edge_pass(2, 64)(src, dst, vsml, _pack_ddt(tbml, 2, 0))
    bml = jnp.concatenate([bmu, bls]).reshape(1, 64)
    outml = _make_combine(64, 2, False)(
        accml, accml, tbml[:, 48:64], vsml, bml)
    return (outml[:, :32], outml[:, 32:])
